# baseline (speedup 1.0000x reference)
import functools

import jax
import jax.numpy as jnp
from jax import lax
from jax.experimental import pallas as pl
from jax.experimental.pallas import tpu as pltpu, tpu_sc as plsc

N_EVENTS = 1_600_000
SIZE = 100_000
NUM_GROUPS = 1_000
S_TIMES = 8
GPAD = 1_024

NC, NS = 2, 16
NW = NC * NS
CHUNK = 800
VPC = CHUNK // 16
UNROLL = 5
LIGHT_TILES = 12
LIGHT_EV = 60 * CHUNK
HEAVY_EV = 64 * CHUNK

_mesh = plsc.VectorSubcoreMesh(
    core_axis_name="c", subcore_axis_name="s", num_cores=NC, num_subcores=NS)


@functools.partial(
    pl.kernel,
    out_type=jax.ShapeDtypeStruct((NW, S_TIMES, GPAD), jnp.float32),
    mesh=_mesh,
    compiler_params=pltpu.CompilerParams(needs_layout_passes=False),
    scratch_types=[
        pltpu.VMEM((SIZE,), jnp.int32),
        [pltpu.VMEM((GPAD,), jnp.float32) for _ in range(S_TIMES)],
        pltpu.VMEM((CHUNK,), jnp.float32),
        pltpu.VMEM((CHUNK,), jnp.float32),
        pltpu.VMEM((CHUNK,), jnp.float32),
        pltpu.VMEM((CHUNK,), jnp.float32),
        pltpu.VMEM((CHUNK,), jnp.float32),
        pltpu.VMEM((CHUNK,), jnp.float32),
        pltpu.VMEM((CHUNK,), jnp.int32),
        pltpu.VMEM((CHUNK,), jnp.int32),
        pltpu.VMEM((16,), jnp.float32),
        pltpu.SemaphoreType.DMA,
        pltpu.SemaphoreType.DMA,
        pltpu.SemaphoreType.DMA,
    ],
)
def _sc_accumulate(rate_h, st_h, en_h, t_h, ix_h, gid_h, out_h,
                   gid_v, accs, rb0, rb1, sb0, sb1, eb0, eb1, ib0, ib1,
                   tv, sem0, sem1, gsem):
    cid = lax.axis_index("c")
    sid = lax.axis_index("s")
    wid = sid * NC + cid
    light = jnp.minimum(wid, LIGHT_TILES)
    heavy = wid - light
    base = light * LIGHT_EV + heavy * HEAVY_EV
    npairs = jnp.where(wid < LIGHT_TILES, 30, 32)

    bufs = [(rb0, sb0, eb0, ib0), (rb1, sb1, eb1, ib1)]
    sems = [sem0, sem1]

    def fire(c, b):
        off = base + c * CHUNK
        rb, sb, eb, ib = bufs[b]
        sem = sems[b]
        pltpu.async_copy(rate_h.at[pl.ds(off, CHUNK)], rb, sem)
        pltpu.async_copy(st_h.at[pl.ds(off, CHUNK)], sb, sem)
        pltpu.async_copy(en_h.at[pl.ds(off, CHUNK)], eb, sem)
        pltpu.async_copy(ix_h.at[pl.ds(off, CHUNK)], ib, sem)

    def wait4(b):
        rb, sb, eb, ib = bufs[b]
        sem = sems[b]
        pltpu.make_async_copy(rate_h.at[pl.ds(0, CHUNK)], rb, sem).wait()
        pltpu.make_async_copy(st_h.at[pl.ds(0, CHUNK)], sb, sem).wait()
        pltpu.make_async_copy(en_h.at[pl.ds(0, CHUNK)], eb, sem).wait()
        pltpu.make_async_copy(ix_h.at[pl.ds(0, CHUNK)], ib, sem).wait()

    fire(0, 0)
    fire(1, 1)

    gid_cp = pltpu.async_copy(gid_h, gid_v, gsem)
    pltpu.sync_copy(t_h, tv.at[pl.ds(0, S_TIMES)])

    zeros16 = jnp.zeros((16,), jnp.float32)

    def zrow(j, carry):
        for s in range(S_TIMES):
            accs[s][pl.ds(j * 16, 16)] = zeros16
        return carry

    lax.fori_loop(0, GPAD // 16, zrow, 0)
    gid_cp.wait()

    tvec = tv[...]
    ts = [tvec[s] for s in range(S_TIMES)]

    def compute(b):
        rb, sb, eb, ib = bufs[b]

        def vec_body(v, inner):
            for u in range(UNROLL):
                o = (v * UNROLL + u) * 16
                r = rb[pl.ds(o, 16)]
                st = sb[pl.ds(o, 16)]
                en = eb[pl.ds(o, 16)]
                ix = ib[pl.ds(o, 16)]
                g = plsc.load_gather(gid_v, [ix])
                for s in range(S_TIMES):
                    m = (st <= ts[s]) & (ts[s] < en)
                    plsc.addupdate_scatter(accs[s], [g], r, mask=m)
            return inner

        lax.fori_loop(0, VPC // UNROLL, vec_body, 0)

    def pair_body(p, carry):
        more = p < npairs - 1
        wait4(0)
        compute(0)

        @pl.when(more)
        def _():
            fire(2 * p + 2, 0)

        wait4(1)
        compute(1)

        @pl.when(more)
        def _():
            fire(2 * p + 3, 1)

        return carry

    lax.fori_loop(0, npairs, pair_body, 0)

    for s in range(S_TIMES):
        pltpu.sync_copy(accs[s], out_h.at[wid, s])


def _sum_body(p_ref, o_ref):
    o_ref[...] = jnp.sum(p_ref[...], axis=0)[:, :NUM_GROUPS]


_tc_sum = pl.pallas_call(
    _sum_body,
    out_shape=jax.ShapeDtypeStruct((S_TIMES, NUM_GROUPS), jnp.float32),
)


def kernel(rate, starttime, endtime, t_points, index, group_ids):
    index = index.astype(jnp.int32)
    group_ids = group_ids.astype(jnp.int32)
    partials = _sc_accumulate(rate, starttime, endtime, t_points,
                              index, group_ids)
    return _tc_sum(partials)


# speedup vs baseline: 1.0149x; 1.0149x over previous
import functools

import jax
import jax.numpy as jnp
from jax import lax
from jax.experimental import pallas as pl
from jax.experimental.pallas import tpu as pltpu, tpu_sc as plsc

N_EVENTS = 1_600_000
SIZE = 100_000
NUM_GROUPS = 1_000
S_TIMES = 8
GPAD = 1_024

NC, NS = 2, 16
NW = NC * NS
CHUNK = 1_600
VPC = CHUNK // 16
UNROLL = 5
LIGHT_TILES = 12
LIGHT_EV = 30 * CHUNK
HEAVY_EV = 32 * CHUNK

_mesh = plsc.VectorSubcoreMesh(
    core_axis_name="c", subcore_axis_name="s", num_cores=NC, num_subcores=NS)


@functools.partial(
    pl.kernel,
    out_type=jax.ShapeDtypeStruct((NW, S_TIMES, GPAD), jnp.float32),
    mesh=_mesh,
    compiler_params=pltpu.CompilerParams(needs_layout_passes=False),
    scratch_types=[
        pltpu.VMEM((SIZE,), jnp.int32),
        [pltpu.VMEM((GPAD,), jnp.float32) for _ in range(S_TIMES)],
        pltpu.VMEM((CHUNK,), jnp.float32),
        pltpu.VMEM((CHUNK,), jnp.float32),
        pltpu.VMEM((CHUNK,), jnp.float32),
        pltpu.VMEM((CHUNK,), jnp.float32),
        pltpu.VMEM((CHUNK,), jnp.float32),
        pltpu.VMEM((CHUNK,), jnp.float32),
        pltpu.VMEM((CHUNK,), jnp.int32),
        pltpu.VMEM((CHUNK,), jnp.int32),
        pltpu.VMEM((16,), jnp.float32),
        pltpu.SemaphoreType.DMA,
        pltpu.SemaphoreType.DMA,
        pltpu.SemaphoreType.DMA,
    ],
)
def _sc_accumulate(rate_h, st_h, en_h, t_h, ix_h, gid_h, out_h,
                   gid_v, accs, rb0, rb1, sb0, sb1, eb0, eb1, ib0, ib1,
                   tv, sem0, sem1, gsem):
    cid = lax.axis_index("c")
    sid = lax.axis_index("s")
    wid = sid * NC + cid
    light = jnp.minimum(wid, LIGHT_TILES)
    heavy = wid - light
    base = light * LIGHT_EV + heavy * HEAVY_EV
    npairs = jnp.where(wid < LIGHT_TILES, 15, 16)

    bufs = [(rb0, sb0, eb0, ib0), (rb1, sb1, eb1, ib1)]
    sems = [sem0, sem1]

    def fire(c, b):
        off = base + c * CHUNK
        rb, sb, eb, ib = bufs[b]
        sem = sems[b]
        pltpu.async_copy(rate_h.at[pl.ds(off, CHUNK)], rb, sem)
        pltpu.async_copy(st_h.at[pl.ds(off, CHUNK)], sb, sem)
        pltpu.async_copy(en_h.at[pl.ds(off, CHUNK)], eb, sem)
        pltpu.async_copy(ix_h.at[pl.ds(off, CHUNK)], ib, sem)

    def wait4(b):
        rb, sb, eb, ib = bufs[b]
        sem = sems[b]
        pltpu.make_async_copy(rate_h.at[pl.ds(0, CHUNK)], rb, sem).wait()
        pltpu.make_async_copy(st_h.at[pl.ds(0, CHUNK)], sb, sem).wait()
        pltpu.make_async_copy(en_h.at[pl.ds(0, CHUNK)], eb, sem).wait()
        pltpu.make_async_copy(ix_h.at[pl.ds(0, CHUNK)], ib, sem).wait()

    fire(0, 0)
    fire(1, 1)

    gid_cp = pltpu.async_copy(gid_h, gid_v, gsem)
    pltpu.sync_copy(t_h, tv.at[pl.ds(0, S_TIMES)])

    zeros16 = jnp.zeros((16,), jnp.float32)

    def zrow(j, carry):
        for s in range(S_TIMES):
            accs[s][pl.ds(j * 16, 16)] = zeros16
        return carry

    lax.fori_loop(0, GPAD // 16, zrow, 0)
    gid_cp.wait()

    tvec = tv[...]
    ts = [tvec[s] for s in range(S_TIMES)]

    def compute(b):
        rb, sb, eb, ib = bufs[b]

        def vec_body(v, inner):
            for u in range(UNROLL):
                o = (v * UNROLL + u) * 16
                r = rb[pl.ds(o, 16)]
                st = sb[pl.ds(o, 16)]
                en = eb[pl.ds(o, 16)]
                ix = ib[pl.ds(o, 16)]
                g = plsc.load_gather(gid_v, [ix])
                for s in range(S_TIMES):
                    m = (st <= ts[s]) & (ts[s] < en)
                    plsc.addupdate_scatter(accs[s], [g], r, mask=m)
            return inner

        lax.fori_loop(0, VPC // UNROLL, vec_body, 0)

    def pair_body(p, carry):
        more = p < npairs - 1
        wait4(0)
        compute(0)

        @pl.when(more)
        def _():
            fire(2 * p + 2, 0)

        wait4(1)
        compute(1)

        @pl.when(more)
        def _():
            fire(2 * p + 3, 1)

        return carry

    lax.fori_loop(0, npairs, pair_body, 0)

    for s in range(S_TIMES):
        pltpu.sync_copy(accs[s], out_h.at[wid, s])


def _sum_body(p_ref, o_ref):
    o_ref[...] = jnp.sum(p_ref[...], axis=0)[:, :NUM_GROUPS]


_tc_sum = pl.pallas_call(
    _sum_body,
    out_shape=jax.ShapeDtypeStruct((S_TIMES, NUM_GROUPS), jnp.float32),
)


def kernel(rate, starttime, endtime, t_points, index, group_ids):
    index = index.astype(jnp.int32)
    group_ids = group_ids.astype(jnp.int32)
    partials = _sc_accumulate(rate, starttime, endtime, t_points,
                              index, group_ids)
    return _tc_sum(partials)
